# baseline (speedup 1.0000x reference)
import functools

import jax
import jax.numpy as jnp
from jax import lax
from jax.experimental import pallas as pl
from jax.experimental.pallas import tpu as pltpu
from jax.experimental.pallas import tpu_sc as plsc

_GROUP_ORDER = 16
_FACTOR = 2
_SUB = _GROUP_ORDER // _FACTOR
_F = 96
_NC = 2
_NS = 16
_NW = _NC * _NS


def kernel(x):
    B, C, H, W = x.shape
    blk = _F * H * W
    nblocks = B * _SUB
    per_w = nblocks // _NW
    xv = x.reshape(nblocks, _FACTOR, blk)

    mesh = plsc.VectorSubcoreMesh(core_axis_name="c", subcore_axis_name="s")

    @functools.partial(
        pl.kernel,
        mesh=mesh,
        out_type=jax.ShapeDtypeStruct((nblocks, blk), jnp.float32),
        scratch_types=[pltpu.SemaphoreType.DMA],
    )
    def sc_copy(x_hbm, out_hbm, sem):
        wid = lax.axis_index("s") * _NC + lax.axis_index("c")
        copies = []
        for k in range(per_w):
            j = wid * per_w + k
            copies.append(pltpu.async_copy(x_hbm.at[j, 0], out_hbm.at[j], sem))
        for cp in copies:
            cp.wait()

    out = sc_copy(xv)
    return out.reshape(B, _SUB * _F, H, W)


# speedup vs baseline: 4.0373x; 4.0373x over previous
import functools

import jax
import jax.numpy as jnp
from jax import lax
from jax.experimental import pallas as pl
from jax.experimental.pallas import tpu as pltpu
from jax.experimental.pallas import tpu_sc as plsc

_GROUP_ORDER = 16
_FACTOR = 2
_SUB = _GROUP_ORDER // _FACTOR
_F = 96
_NC = 2
_NS = 16
_NW = _NC * _NS

_CHUNK = 32768
_NBUF = 3


def kernel(x):
    B, C, H, W = x.shape
    blk = _F * H * W
    nblocks = B * _SUB
    per_w = nblocks // _NW
    cpb = blk // _CHUNK
    nchunks = per_w * cpb
    xv = x.reshape(nblocks, _FACTOR, blk)

    mesh = plsc.VectorSubcoreMesh(core_axis_name="c", subcore_axis_name="s")

    @functools.partial(
        pl.kernel,
        mesh=mesh,
        out_type=jax.ShapeDtypeStruct((nblocks, blk), jnp.float32),
        scratch_types=(
            [pltpu.VMEM((_CHUNK,), jnp.float32)] * _NBUF
            + [pltpu.SemaphoreType.DMA] * (2 * _NBUF)
        ),
    )
    def sc_copy(x_hbm, out_hbm, *scratch):
        bufs = scratch[:_NBUF]
        insems = scratch[_NBUF:2 * _NBUF]
        outsems = scratch[2 * _NBUF:3 * _NBUF]
        wid = lax.axis_index("s") * _NC + lax.axis_index("c")

        def src(i):
            j = wid * per_w + i // cpb
            return x_hbm.at[j, 0, pl.ds((i % cpb) * _CHUNK, _CHUNK)]

        def dst(i):
            j = wid * per_w + i // cpb
            return out_hbm.at[j, pl.ds((i % cpb) * _CHUNK, _CHUNK)]

        in_cp = [None] * nchunks
        out_cp = [None] * nchunks
        for t in range(nchunks + 1):
            if t < nchunks:
                b = t % _NBUF
                if t >= _NBUF:
                    out_cp[t - _NBUF].wait()
                in_cp[t] = pltpu.async_copy(src(t), bufs[b], insems[b])
            if t >= 1:
                i = t - 1
                b = i % _NBUF
                in_cp[i].wait()
                out_cp[i] = pltpu.async_copy(bufs[b], dst(i), outsems[b])
        for i in range(nchunks - _NBUF, nchunks):
            out_cp[i].wait()

    out = sc_copy(xv)
    return out.reshape(B, _SUB * _F, H, W)


# speedup vs baseline: 8.4755x; 2.0993x over previous
import jax
import jax.numpy as jnp
from jax.experimental import pallas as pl

_GROUP_ORDER = 16
_FACTOR = 2
_SUB = _GROUP_ORDER // _FACTOR
_F = 96


def _copy_body(in_ref, out_ref):
    out_ref[...] = in_ref[...]


def kernel(x):
    B, C, H, W = x.shape
    xv = x.reshape(B, _GROUP_ORDER, _F, H, W)

    split = 4
    fs = _F // split
    out = pl.pallas_call(
        _copy_body,
        grid=(B, _SUB, split),
        in_specs=[
            pl.BlockSpec((1, 1, fs, H, W), lambda b, g, j: (b, _FACTOR * g, j, 0, 0))
        ],
        out_specs=pl.BlockSpec((1, 1, fs, H, W), lambda b, g, j: (b, g, j, 0, 0)),
        out_shape=jax.ShapeDtypeStruct((B, _SUB, _F, H, W), jnp.float32),
    )(xv)
    return out.reshape(B, _SUB * _F, H, W)
